# baseline (speedup 1.0000x reference)
import functools

import jax
import jax.numpy as jnp
from jax import lax
from jax.experimental import pallas as pl
from jax.experimental.pallas import tpu as pltpu
from jax.experimental.pallas import tpu_sc as plsc

PAD = 0
B, H, D = 4096, 200, 32
L = 16
NG = 13
HP = NG * L
HPAD = 256
ROWS_PER_CHUNK = 8


def _fused_sc(idx_tensor, table, w):
    info = plsc.get_sparse_core_info()
    nc = info.num_cores
    nw = nc * info.num_subcores
    rows_per_w = B // nw
    n_chunks = rows_per_w // ROWS_PER_CHUNK
    mesh = plsc.VectorSubcoreMesh(core_axis_name="c", subcore_axis_name="s")

    @functools.partial(
        pl.kernel,
        mesh=mesh,
        out_type=jax.ShapeDtypeStruct((B, D), jnp.float32),
        scratch_types=[
            pltpu.VMEM((ROWS_PER_CHUNK * HP,), jnp.int32),
            pltpu.VMEM((ROWS_PER_CHUNK * HP, D), jnp.float32),
            pltpu.VMEM((1, D), jnp.float32),
            pltpu.VMEM((ROWS_PER_CHUNK, D), jnp.float32),
            pltpu.SemaphoreType.DMA,
            pltpu.SemaphoreType.DMA,
        ],
        compiler_params=pltpu.CompilerParams(use_tc_tiling_on_sc=False),
    )
    def k(idx_hbm, table_hbm, w_hbm, out_hbm, idx_v, rows_v, w_v, stage_v,
          sem_i, sem_g):
        wid = lax.axis_index("s") * nc + lax.axis_index("c")
        row_base = wid * rows_per_w
        pltpu.sync_copy(w_hbm, w_v)
        lane = lax.iota(jnp.int32, L)
        w_lo = w_v[0, pl.ds(0, L)]
        w_hi = w_v[0, pl.ds(L, L)]
        perms = [lane ^ k for k in (1, 2, 4, 8)]
        _dnums = lax.GatherDimensionNumbers(
            offset_dims=(), collapsed_slice_dims=(0,), start_index_map=(0,))

        def _perm(u, p):
            return lax.gather(u, p[:, None], _dnums, (1,),
                              mode=lax.GatherScatterMode.PROMISE_IN_BOUNDS)

        def _allsum(u):
            for p in perms:
                u = u + _perm(u, p)
            return u

        def _allmax(u):
            for p in perms:
                u = jnp.maximum(u, _perm(u, p))
            return u

        def do_chunk(c, carry):
            row0 = row_base + c * ROWS_PER_CHUNK
            cps = [
                pltpu.async_copy(idx_hbm.at[row0 + i, pl.ds(0, HP)],
                                 idx_v.at[pl.ds(i * HP, HP)], sem_i)
                for i in range(ROWS_PER_CHUNK)
            ]
            for cp in cps:
                cp.wait()
            cps = [
                pltpu.async_copy(table_hbm.at[idx_v.at[pl.ds(i * HP, H)]],
                                 rows_v.at[pl.ds(i * HP, H)], sem_g)
                for i in range(ROWS_PER_CHUNK)
            ]
            for cp in cps:
                cp.wait()

            def do_row(r, rcarry):
                rb = r * HP
                attn = []
                for g in range(NG):
                    ib = rb + g * L
                    sv = jnp.zeros((L,), jnp.float32)
                    for j in range(L):
                        i = ib + j
                        v_lo = rows_v[i, pl.ds(0, L)]
                        v_hi = rows_v[i, pl.ds(L, L)]
                        s = _allsum(v_lo * w_lo + v_hi * w_hi)
                        sv = jnp.where(lane == j, s, sv)
                    idxg = idx_v[pl.ds(ib, L)]
                    valid = idxg != PAD
                    if g == NG - 1:
                        valid = valid & (lane < H - (NG - 1) * L)
                    attn.append(jnp.where(valid, sv, jnp.float32(-1e30)))
                m = attn[0]
                for g in range(1, NG):
                    m = jnp.maximum(m, attn[g])
                mm = _allmax(m)
                attn = [jnp.exp(a - mm) for a in attn]
                z = attn[0]
                for g in range(1, NG):
                    z = z + attn[g]
                inv = jnp.float32(1.0) / _allsum(z)
                attn = [a * inv for a in attn]
                acc_lo = jnp.zeros((L,), jnp.float32)
                acc_hi = jnp.zeros((L,), jnp.float32)
                for g in range(NG):
                    ib = rb + g * L
                    a = attn[g]
                    jmax = L if g < NG - 1 else H - (NG - 1) * L
                    for j in range(jmax):
                        i = ib + j
                        aj = _perm(a, jnp.full((L,), j, jnp.int32))
                        acc_lo = acc_lo + rows_v[i, pl.ds(0, L)] * aj
                        acc_hi = acc_hi + rows_v[i, pl.ds(L, L)] * aj
                stage_v[r, pl.ds(0, L)] = acc_lo
                stage_v[r, pl.ds(L, L)] = acc_hi
                return rcarry

            lax.fori_loop(0, ROWS_PER_CHUNK, do_row, 0)
            pltpu.sync_copy(stage_v, out_hbm.at[pl.ds(row0, ROWS_PER_CHUNK)])
            return carry

        lax.fori_loop(0, n_chunks, do_chunk, 0)

    return k(idx_tensor, table, w)


def kernel(idx_tensor, table, attn_weight, attn_bias):
    del attn_bias
    idx_p = jnp.pad(idx_tensor, ((0, 0), (0, HPAD - H)))
    return _fused_sc(idx_p, table, attn_weight)


# speedup vs baseline: 1.1715x; 1.1715x over previous
import functools

import jax
import jax.numpy as jnp
from jax import lax
from jax.experimental import pallas as pl
from jax.experimental.pallas import tpu as pltpu
from jax.experimental.pallas import tpu_sc as plsc

PAD = 0
B, H, D = 4096, 200, 32
BH = B * H
HD = H * D


HPAD = 256
RPC = 8


def _sc_gather(idx_p, table):
    info = plsc.get_sparse_core_info()
    nw = info.num_cores * info.num_subcores
    rows_per_w = B // nw
    n_ch = rows_per_w // RPC
    ch = RPC * H
    mesh = plsc.VectorSubcoreMesh(core_axis_name="c", subcore_axis_name="s")

    @functools.partial(
        pl.kernel,
        mesh=mesh,
        out_type=jax.ShapeDtypeStruct((BH, D), jnp.float32),
        scratch_types=[
            pltpu.VMEM((RPC * HPAD,), jnp.int32),
            pltpu.VMEM((ch, D), jnp.float32),
            pltpu.SemaphoreType.DMA,
            pltpu.SemaphoreType.DMA,
        ],
        compiler_params=pltpu.CompilerParams(use_tc_tiling_on_sc=False),
    )
    def k(idx_hbm, table_hbm, out_hbm, idx_v, rows_v, sem_i, sem_g):
        wid = lax.axis_index("s") * info.num_cores + lax.axis_index("c")
        base = wid * rows_per_w

        def body(c, carry):
            row0 = base + c * RPC
            cps = [
                pltpu.async_copy(idx_hbm.at[row0 + i, pl.ds(0, HPAD)],
                                 idx_v.at[pl.ds(i * HPAD, HPAD)], sem_i)
                for i in range(RPC)
            ]
            for cp in cps:
                cp.wait()
            cps = [
                pltpu.async_copy(table_hbm.at[idx_v.at[pl.ds(i * HPAD, H)]],
                                 rows_v.at[pl.ds(i * H, H)], sem_g)
                for i in range(RPC)
            ]
            for cp in cps:
                cp.wait()
            pltpu.sync_copy(rows_v, out_hbm.at[pl.ds(row0 * H, ch)])
            return carry

        lax.fori_loop(0, n_ch, body, 0)

    return k(idx_p, table)


def _tc_compute(idx, e2, w_sel, e_exp, r_sel):
    bc = 128
    grid = (B // bc,)

    def body(idx_ref, e_ref, ws_ref, ee_ref, r_ref, out_ref):
        idxb = idx_ref[...]
        e = e_ref[...]
        scores = jnp.dot(e, ws_ref[...], preferred_element_type=jnp.float32)
        valid = idxb != PAD
        has_real = jnp.any(valid, axis=1, keepdims=True)
        col = lax.broadcasted_iota(jnp.int32, (bc, H), 1)
        valid = valid | ((col == 0) & jnp.logical_not(has_real))
        scores = jnp.where(valid, scores, -jnp.inf)
        m = jnp.max(scores, axis=1, keepdims=True)
        p = jnp.exp(scores - m)
        z = jnp.sum(p, axis=1, keepdims=True)
        attn = p / z
        af = jnp.dot(attn, ee_ref[...], preferred_element_type=jnp.float32)
        out_ref[...] = jnp.dot(af * e, r_ref[...],
                               preferred_element_type=jnp.float32)

    return pl.pallas_call(
        body,
        grid=grid,
        in_specs=[
            pl.BlockSpec((bc, H), lambda i: (i, 0)),
            pl.BlockSpec((bc, HD), lambda i: (i, 0)),
            pl.BlockSpec((HD, H), lambda i: (0, 0)),
            pl.BlockSpec((H, HD), lambda i: (0, 0)),
            pl.BlockSpec((HD, D), lambda i: (0, 0)),
        ],
        out_specs=pl.BlockSpec((bc, D), lambda i: (i, 0)),
        out_shape=jax.ShapeDtypeStruct((B, D), jnp.float32),
    )(idx, e2, w_sel, e_exp, r_sel)


def _weight_mats(w):
    flat = jnp.arange(HD, dtype=jnp.int32)
    item = flat // D
    dim = flat % D
    items = jnp.arange(H, dtype=jnp.int32)
    dims = jnp.arange(D, dtype=jnp.int32)
    wt = jnp.tile(w, H)
    w_sel = jnp.where(item[:, None] == items[None, :], wt[:, None], 0.0)
    e_exp = (items[:, None] == item[None, :]).astype(jnp.float32)
    r_sel = (dim[:, None] == dims[None, :]).astype(jnp.float32)
    return w_sel, e_exp, r_sel


def kernel(idx_tensor, table, attn_weight, attn_bias):
    del attn_bias
    idx_p = jnp.pad(idx_tensor, ((0, 0), (0, HPAD - H)))
    embs = _sc_gather(idx_p, table)
    e2 = embs.reshape(B, HD)
    w_sel, e_exp, r_sel = _weight_mats(attn_weight.reshape(D))
    return _tc_compute(idx_tensor, e2, w_sel, e_exp, r_sel)


# speedup vs baseline: 1.1751x; 1.0030x over previous
import functools

import jax
import jax.numpy as jnp
from jax import lax
from jax.experimental import pallas as pl
from jax.experimental.pallas import tpu as pltpu
from jax.experimental.pallas import tpu_sc as plsc

PAD = 0
B, H, D = 4096, 200, 32
BH = B * H
HD = H * D


HPAD = 256
RPC = 8


def _sc_gather(idx_p, table):
    info = plsc.get_sparse_core_info()
    nw = info.num_cores * info.num_subcores
    rows_per_w = B // nw
    n_ch = rows_per_w // RPC
    ch = RPC * H
    mesh = plsc.VectorSubcoreMesh(core_axis_name="c", subcore_axis_name="s")

    @functools.partial(
        pl.kernel,
        mesh=mesh,
        out_type=jax.ShapeDtypeStruct((BH, D), jnp.float32),
        scratch_types=[
            pltpu.VMEM((RPC * HPAD,), jnp.int32),
            pltpu.VMEM((ch, D), jnp.float32),
            pltpu.SemaphoreType.DMA,
            pltpu.SemaphoreType.DMA,
        ],
        compiler_params=pltpu.CompilerParams(use_tc_tiling_on_sc=False),
    )
    def k(idx_hbm, table_hbm, out_hbm, idx_v, rows_v, sem_i, sem_g):
        wid = lax.axis_index("s") * info.num_cores + lax.axis_index("c")
        base = wid * rows_per_w

        def body(c, carry):
            row0 = base + c * RPC
            cps = [
                pltpu.async_copy(idx_hbm.at[row0 + i, pl.ds(0, HPAD)],
                                 idx_v.at[pl.ds(i * HPAD, HPAD)], sem_i)
                for i in range(RPC)
            ]
            for cp in cps:
                cp.wait()
            cps = [
                pltpu.async_copy(table_hbm.at[idx_v.at[pl.ds(i * HPAD, H)]],
                                 rows_v.at[pl.ds(i * H, H)], sem_g)
                for i in range(RPC)
            ]
            for cp in cps:
                cp.wait()
            pltpu.sync_copy(rows_v, out_hbm.at[pl.ds(row0 * H, ch)])
            return carry

        lax.fori_loop(0, n_ch, body, 0)

    return k(idx_p, table)


def _tc_compute(idx, e2, w_sel, e_exp, r_sel):
    bc = 128
    grid = (B // bc,)

    def body(idx_ref, e_ref, ws_ref, ee_ref, r_ref, out_ref):
        idxb = idx_ref[...]
        e = e_ref[...]
        eb = e.astype(jnp.bfloat16)
        scores = jnp.dot(eb, ws_ref[...], preferred_element_type=jnp.float32)
        valid = idxb != PAD
        has_real = jnp.any(valid, axis=1, keepdims=True)
        col = lax.broadcasted_iota(jnp.int32, (bc, H), 1)
        valid = valid | ((col == 0) & jnp.logical_not(has_real))
        scores = jnp.where(valid, scores, -jnp.inf)
        m = jnp.max(scores, axis=1, keepdims=True)
        p = jnp.exp(scores - m)
        z = jnp.sum(p, axis=1, keepdims=True)
        attn = p / z
        af = jnp.dot(attn.astype(jnp.bfloat16), ee_ref[...],
                     preferred_element_type=jnp.float32)
        out_ref[...] = jnp.dot((af * e).astype(jnp.bfloat16), r_ref[...],
                               preferred_element_type=jnp.float32)

    return pl.pallas_call(
        body,
        grid=grid,
        in_specs=[
            pl.BlockSpec((bc, H), lambda i: (i, 0)),
            pl.BlockSpec((bc, HD), lambda i: (i, 0)),
            pl.BlockSpec((HD, H), lambda i: (0, 0)),
            pl.BlockSpec((H, HD), lambda i: (0, 0)),
            pl.BlockSpec((HD, D), lambda i: (0, 0)),
        ],
        out_specs=pl.BlockSpec((bc, D), lambda i: (i, 0)),
        out_shape=jax.ShapeDtypeStruct((B, D), jnp.float32),
    )(idx, e2, w_sel, e_exp, r_sel)


def _weight_mats(w):
    flat = jnp.arange(HD, dtype=jnp.int32)
    item = flat // D
    dim = flat % D
    items = jnp.arange(H, dtype=jnp.int32)
    dims = jnp.arange(D, dtype=jnp.int32)
    wt = jnp.tile(w, H)
    w_sel = jnp.where(item[:, None] == items[None, :], wt[:, None],
                      0.0).astype(jnp.bfloat16)
    e_exp = (items[:, None] == item[None, :]).astype(jnp.bfloat16)
    r_sel = (dim[:, None] == dims[None, :]).astype(jnp.bfloat16)
    return w_sel, e_exp, r_sel


def kernel(idx_tensor, table, attn_weight, attn_bias):
    del attn_bias
    idx_p = jnp.pad(idx_tensor, ((0, 0), (0, HPAD - H)))
    embs = _sc_gather(idx_p, table)
    e2 = embs.reshape(B, HD)
    w_sel, e_exp, r_sel = _weight_mats(attn_weight.reshape(D))
    return _tc_compute(idx_tensor, e2, w_sel, e_exp, r_sel)
